# baseline (speedup 1.0000x reference)
import functools

import jax
import jax.numpy as jnp
from jax import lax
from jax.experimental import pallas as pl
from jax.experimental.pallas import tpu as pltpu
from jax.experimental.pallas import tpu_sc as plsc

BATCH = 16384
NFIX = 100

_NC = 2
_NS = 16
_NW = _NC * _NS
_BPW = BATCH // _NW
_CHUNK = 128
_NCHUNK = _BPW // _CHUNK


def _sc_gather_body(idx_hbm, table_hbm, out_hbm, idx_v, vals_v, sem):
    wid = lax.axis_index("c") * _NS + lax.axis_index("s")
    base = wid * _BPW
    for j in range(_NCHUNK):
        pltpu.sync_copy(idx_hbm.at[pl.ds(base + j * _CHUNK, _CHUNK)],
                        idx_v.at[j])
    copies = [
        pltpu.async_copy(table_hbm.at[idx_v.at[j]], vals_v.at[j], sem)
        for j in range(_NCHUNK)
    ]
    for c in copies:
        c.wait()
    for j in range(_NCHUNK):
        pltpu.sync_copy(vals_v.at[j],
                        out_hbm.at[pl.ds(base + j * _CHUNK, _CHUNK)])


def _sc_gather(idx, table):
    mesh = plsc.VectorSubcoreMesh(core_axis_name="c", subcore_axis_name="s")
    return pl.kernel(
        _sc_gather_body,
        out_type=jax.ShapeDtypeStruct((BATCH,), jnp.float32),
        mesh=mesh,
        scratch_types=[
            pltpu.VMEM((_NCHUNK, _CHUNK), jnp.int32),
            pltpu.VMEM((_NCHUNK, _CHUNK), jnp.float32),
            pltpu.SemaphoreType.DMA,
        ],
    )(idx, table)


_BLK = 2048


def _tc_body(x_ref, w_ref, b_ref, r_ref, o_ref):
    x = x_ref[...]
    w = w_ref[...]
    s = jnp.sum(x * w, axis=1)
    o_ref[...] = s + r_ref[...] + b_ref[0]


def kernel(X_fixed, sample_indices, W, b, random_effects):
    r = _sc_gather(sample_indices, random_effects)
    out = pl.pallas_call(
        _tc_body,
        grid=(BATCH // _BLK,),
        in_specs=[
            pl.BlockSpec((_BLK, NFIX), lambda i: (i, 0)),
            pl.BlockSpec((1, NFIX), lambda i: (0, 0)),
            pl.BlockSpec(memory_space=pltpu.SMEM),
            pl.BlockSpec((_BLK,), lambda i: (i,)),
        ],
        out_specs=pl.BlockSpec((_BLK,), lambda i: (i,)),
        out_shape=jax.ShapeDtypeStruct((BATCH,), jnp.float32),
    )(X_fixed, W, b, r)
    return out


# speedup vs baseline: 1.1474x; 1.1474x over previous
import functools

import jax
import jax.numpy as jnp
from jax import lax
from jax.experimental import pallas as pl
from jax.experimental.pallas import tpu as pltpu
from jax.experimental.pallas import tpu_sc as plsc

BATCH = 16384
NFIX = 100

_NC = 2
_NS = 16
_NW = _NC * _NS
_BPW = BATCH // _NW
_CHUNK = 128
_NCHUNK = _BPW // _CHUNK


def _sc_gather_body(idx_hbm, table_hbm, out_hbm, idx_v, vals_v, sem):
    wid = lax.axis_index("c") * _NS + lax.axis_index("s")
    base = wid * _BPW
    pltpu.sync_copy(idx_hbm.at[pl.ds(base, _BPW)], idx_v)
    copies = [
        pltpu.async_copy(table_hbm.at[idx_v.at[pl.ds(j * _CHUNK, _CHUNK)]],
                         vals_v.at[pl.ds(j * _CHUNK, _CHUNK)], sem)
        for j in range(_NCHUNK)
    ]
    for c in copies:
        c.wait()
    pltpu.sync_copy(vals_v, out_hbm.at[pl.ds(base, _BPW)])


def _sc_gather(idx, table):
    mesh = plsc.VectorSubcoreMesh(core_axis_name="c", subcore_axis_name="s")
    return pl.kernel(
        _sc_gather_body,
        out_type=jax.ShapeDtypeStruct((BATCH,), jnp.float32),
        mesh=mesh,
        scratch_types=[
            pltpu.VMEM((_BPW,), jnp.int32),
            pltpu.VMEM((_BPW,), jnp.float32),
            pltpu.SemaphoreType.DMA,
        ],
    )(idx, table)


_BLK = 2048


def _tc_body(x_ref, w_ref, b_ref, r_ref, o_ref):
    x = x_ref[...]
    w = w_ref[...]
    s = jax.lax.dot_general(w, x, (((1,), (1,)), ((), ())),
                            preferred_element_type=jnp.float32)
    o_ref[...] = s + r_ref[...] + b_ref[0]


def kernel(X_fixed, sample_indices, W, b, random_effects):
    r = _sc_gather(sample_indices, random_effects)
    out = pl.pallas_call(
        _tc_body,
        grid=(BATCH // _BLK,),
        in_specs=[
            pl.BlockSpec((_BLK, NFIX), lambda i: (i, 0)),
            pl.BlockSpec((1, NFIX), lambda i: (0, 0)),
            pl.BlockSpec(memory_space=pltpu.SMEM),
            pl.BlockSpec((1, _BLK), lambda i: (0, i)),
        ],
        out_specs=pl.BlockSpec((1, _BLK), lambda i: (0, i)),
        out_shape=jax.ShapeDtypeStruct((1, BATCH), jnp.float32),
    )(X_fixed, W, b, r.reshape(1, BATCH))
    return out.reshape(BATCH)


# speedup vs baseline: 1.5497x; 1.3506x over previous
import jax
import jax.numpy as jnp
from jax import lax
from jax.experimental import pallas as pl
from jax.experimental.pallas import tpu as pltpu
from jax.experimental.pallas import tpu_sc as plsc

BATCH = 16384
NFIX = 100

_NC = 2
_NS = 16
_NW = _NC * _NS
_BPW = BATCH // _NW
_CHUNK = 128
_NCHUNK = _BPW // _CHUNK


def _sc_gather_body(idx_hbm, table_hbm, out_hbm, idx_v, vals_v, isems, gsem):
    wid = lax.axis_index("c") * _NS + lax.axis_index("s")
    base = wid * _BPW
    idx_copies = [
        pltpu.async_copy(idx_hbm.at[pl.ds(base + j * _CHUNK, _CHUNK)],
                         idx_v.at[pl.ds(j * _CHUNK, _CHUNK)], isems.at[j])
        for j in range(_NCHUNK)
    ]
    gathers = []
    for j in range(_NCHUNK):
        idx_copies[j].wait()
        gathers.append(
            pltpu.async_copy(
                table_hbm.at[idx_v.at[pl.ds(j * _CHUNK, _CHUNK)]],
                vals_v.at[pl.ds(j * _CHUNK, _CHUNK)], gsem))
    for g in gathers:
        g.wait()
    pltpu.sync_copy(vals_v, out_hbm.at[pl.ds(base, _BPW)])


def _sc_gather(idx, table):
    mesh = plsc.VectorSubcoreMesh(core_axis_name="c", subcore_axis_name="s")
    return pl.kernel(
        _sc_gather_body,
        out_type=jax.ShapeDtypeStruct((BATCH,), jnp.float32),
        mesh=mesh,
        scratch_types=[
            pltpu.VMEM((_BPW,), jnp.int32),
            pltpu.VMEM((_BPW,), jnp.float32),
            pltpu.SemaphoreType.DMA((_NCHUNK,)),
            pltpu.SemaphoreType.DMA,
        ],
    )(idx, table)


_BLK = 2048


def _mv_body(xt_ref, w_ref, b_ref, o_ref):
    s = lax.dot_general(w_ref[...], xt_ref[...], (((1,), (0,)), ((), ())),
                        preferred_element_type=jnp.float32)
    o_ref[...] = s + b_ref[0]


def _add_body(f_ref, r_ref, o_ref):
    o_ref[...] = f_ref[...] + r_ref[...]


def kernel(X_fixed, sample_indices, W, b, random_effects):
    r = _sc_gather(sample_indices, random_effects)
    xt = X_fixed.T
    fixed = pl.pallas_call(
        _mv_body,
        grid=(BATCH // _BLK,),
        in_specs=[
            pl.BlockSpec((NFIX, _BLK), lambda i: (0, i)),
            pl.BlockSpec((1, NFIX), lambda i: (0, 0)),
            pl.BlockSpec(memory_space=pltpu.SMEM),
        ],
        out_specs=pl.BlockSpec((1, _BLK), lambda i: (0, i)),
        out_shape=jax.ShapeDtypeStruct((1, BATCH), jnp.float32),
    )(xt, W, b)
    out = pl.pallas_call(
        _add_body,
        in_specs=[
            pl.BlockSpec((1, BATCH), lambda: (0, 0)),
            pl.BlockSpec((1, BATCH), lambda: (0, 0)),
        ],
        out_specs=pl.BlockSpec((1, BATCH), lambda: (0, 0)),
        out_shape=jax.ShapeDtypeStruct((1, BATCH), jnp.float32),
    )(fixed, r.reshape(1, BATCH))
    return out.reshape(BATCH)


# speedup vs baseline: 1.6137x; 1.0413x over previous
import jax
import jax.numpy as jnp
from jax import lax
from jax.experimental import pallas as pl
from jax.experimental.pallas import tpu as pltpu
from jax.experimental.pallas import tpu_sc as plsc

BATCH = 16384
NFIX = 100

_NC = 2
_NS = 16
_NW = _NC * _NS
_BPW = BATCH // _NW
_CHUNK = 128
_NCHUNK = _BPW // _CHUNK


def _sc_gather_body(idx_hbm, table_hbm, out_hbm, idx_v, vals_v, isems, gsem):
    wid = lax.axis_index("c") * _NS + lax.axis_index("s")
    base = wid * _BPW
    idx_copies = [
        pltpu.async_copy(idx_hbm.at[pl.ds(base + j * _CHUNK, _CHUNK)],
                         idx_v.at[pl.ds(j * _CHUNK, _CHUNK)], isems.at[j])
        for j in range(_NCHUNK)
    ]
    gathers = []
    for j in range(_NCHUNK):
        idx_copies[j].wait()
        gathers.append(
            pltpu.async_copy(
                table_hbm.at[idx_v.at[pl.ds(j * _CHUNK, _CHUNK)]],
                vals_v.at[pl.ds(j * _CHUNK, _CHUNK)], isems.at[j]))
    wbs = []
    for j in range(_NCHUNK):
        gathers[j].wait()
        wbs.append(
            pltpu.async_copy(vals_v.at[pl.ds(j * _CHUNK, _CHUNK)],
                             out_hbm.at[pl.ds(base + j * _CHUNK, _CHUNK)],
                             gsem))
    for w in wbs:
        w.wait()


def _sc_gather(idx, table):
    mesh = plsc.VectorSubcoreMesh(core_axis_name="c", subcore_axis_name="s")
    return pl.kernel(
        _sc_gather_body,
        out_type=jax.ShapeDtypeStruct((BATCH,), jnp.float32),
        mesh=mesh,
        scratch_types=[
            pltpu.VMEM((_BPW,), jnp.int32),
            pltpu.VMEM((_BPW,), jnp.float32),
            pltpu.SemaphoreType.DMA((_NCHUNK,)),
            pltpu.SemaphoreType.DMA,
        ],
    )(idx, table)


_BLK = 8192


def _mv_body(xt_ref, w_ref, b_ref, o_ref):
    s = lax.dot_general(w_ref[...], xt_ref[...], (((1,), (0,)), ((), ())),
                        preferred_element_type=jnp.float32)
    o_ref[...] = s + b_ref[0]


def _add_body(f_ref, r_ref, o_ref):
    o_ref[...] = f_ref[...] + r_ref[...]


def kernel(X_fixed, sample_indices, W, b, random_effects):
    r = _sc_gather(sample_indices, random_effects)
    xt = X_fixed.T
    fixed = pl.pallas_call(
        _mv_body,
        grid=(BATCH // _BLK,),
        in_specs=[
            pl.BlockSpec((NFIX, _BLK), lambda i: (0, i)),
            pl.BlockSpec((1, NFIX), lambda i: (0, 0)),
            pl.BlockSpec(memory_space=pltpu.SMEM),
        ],
        out_specs=pl.BlockSpec((1, _BLK), lambda i: (0, i)),
        out_shape=jax.ShapeDtypeStruct((1, BATCH), jnp.float32),
    )(xt, W, b)
    out = pl.pallas_call(
        _add_body,
        in_specs=[
            pl.BlockSpec((1, BATCH), lambda: (0, 0)),
            pl.BlockSpec((1, BATCH), lambda: (0, 0)),
        ],
        out_specs=pl.BlockSpec((1, BATCH), lambda: (0, 0)),
        out_shape=jax.ShapeDtypeStruct((1, BATCH), jnp.float32),
    )(fixed, r.reshape(1, BATCH))
    return out.reshape(BATCH)
